# baseline (speedup 1.0000x reference)
import functools

import jax
import jax.numpy as jnp
from jax import lax
from jax.experimental import pallas as pl
from jax.experimental.pallas import tpu as pltpu
from jax.experimental.pallas import tpu_sc as plsc

N_ACTIONS = 18
EPS = 0.99
LO = EPS / N_ACTIONS
HI = 1.0 - EPS + EPS / N_ACTIONS

L = 16
NC, NS = 2, 16
NW = NC * NS
IDX_CHUNK = 128


def kernel(obs, q_table):
    B = obs.shape[0]
    A = q_table.shape[1]
    b_per_w = B // NW
    n_chunks = b_per_w // IDX_CHUNK
    n_groups = b_per_w // L

    mesh = plsc.VectorSubcoreMesh(core_axis_name="c", subcore_axis_name="s")

    @functools.partial(
        pl.kernel,
        out_type=jax.ShapeDtypeStruct((B, A), jnp.float32),
        mesh=mesh,
        scratch_types=[
            pltpu.VMEM((n_chunks, IDX_CHUNK), jnp.int32),
            pltpu.VMEM((b_per_w, A), jnp.float32),
            pltpu.VMEM((b_per_w, A), jnp.float32),
            pltpu.SemaphoreType.DMA,
        ],
        compiler_params=pltpu.CompilerParams(
            needs_layout_passes=False, use_tc_tiling_on_sc=False
        ),
    )
    def qpolicy(obs_hbm, q_hbm, out_hbm, idx_v, rows_v, out_v, sem):
        wid = lax.axis_index("s") * NC + lax.axis_index("c")
        base = wid * b_per_w

        for j in range(n_chunks):
            pltpu.sync_copy(
                obs_hbm.at[pl.ds(base + j * IDX_CHUNK, IDX_CHUNK)],
                idx_v.at[j],
            )
        copies = [
            pltpu.async_copy(
                q_hbm.at[idx_v.at[j]],
                rows_v.at[pl.ds(j * IDX_CHUNK, IDX_CHUNK)],
                sem,
            )
            for j in range(n_chunks)
        ]
        for c in copies:
            c.wait()

        lo = jnp.full((L,), LO, jnp.float32)
        hi = jnp.full((L,), HI, jnp.float32)
        lane = lax.iota(jnp.int32, L)

        def group_body(g, carry):
            rows = lane + g * L
            zero = jnp.zeros((L,), jnp.int32)
            best = plsc.load_gather(rows_v, [rows, zero])
            besta = zero
            for a in range(1, A):
                ca = jnp.full((L,), a, jnp.int32)
                va = plsc.load_gather(rows_v, [rows, ca])
                m = va > best
                best = jnp.where(m, va, best)
                besta = jnp.where(m, ca, besta)
            for a in range(A):
                ca = jnp.full((L,), a, jnp.int32)
                vals = jnp.where(besta == ca, hi, lo)
                plsc.store_scatter(out_v, [rows, ca], vals)
            return carry

        lax.fori_loop(0, n_groups, group_body, 0)
        pltpu.sync_copy(out_v, out_hbm.at[pl.ds(base, b_per_w)])

    return qpolicy(obs, q_table)


# speedup vs baseline: 11.1908x; 11.1908x over previous
import functools

import jax
import jax.numpy as jnp
from jax import lax
from jax.experimental import pallas as pl
from jax.experimental.pallas import tpu as pltpu
from jax.experimental.pallas import tpu_sc as plsc

N_ACTIONS = 18
EPS = 0.99
LO = EPS / N_ACTIONS
HI = 1.0 - EPS + EPS / N_ACTIONS

L = 16
NC, NS = 2, 16
NW = NC * NS
NSLOT = 16


def kernel(obs, q_table):
    B = obs.shape[0]
    A = q_table.shape[1]
    b_per_w = B // NW
    n_groups = b_per_w // L
    n_pairs = n_groups // 2

    qT = q_table.T

    mesh = plsc.VectorSubcoreMesh(core_axis_name="c", subcore_axis_name="s")

    @functools.partial(
        pl.kernel,
        out_type=jax.ShapeDtypeStruct((A, B), jnp.float32),
        mesh=mesh,
        scratch_types=[
            pltpu.VMEM((b_per_w // L, L), jnp.int32),
            pltpu.VMEM((2 * NSLOT * A, 128), jnp.float32),
            pltpu.VMEM((A, b_per_w), jnp.float32),
            pltpu.SemaphoreType.DMA,
            pltpu.SemaphoreType.DMA,
        ],
        compiler_params=pltpu.CompilerParams(
            needs_layout_passes=False, use_tc_tiling_on_sc=True
        ),
    )
    def qpolicy(obs_hbm, qT_hbm, outT_hbm, obs_v, slots_v, out_v,
                sem0, sem1):
        wid = lax.axis_index("s") * NC + lax.axis_index("c")
        base = pl.multiple_of(wid * b_per_w, 128)

        for r in range(b_per_w // L):
            pltpu.sync_copy(
                obs_hbm.at[pl.ds(base + r * L, L)], obs_v.at[r]
            )

        def fire(g, buf, sem):
            iv = obs_v[g, :]
            for k in range(NSLOT):
                i = iv[k]
                tile0 = pl.multiple_of((i // 128) * 128, 128)
                pltpu.async_copy(
                    qT_hbm.at[:, pl.ds(tile0, 128)],
                    slots_v.at[pl.ds((buf * NSLOT + k) * A, A)],
                    sem,
                )

        def drain(buf, sem):
            for k in range(NSLOT):
                pltpu.make_async_copy(
                    qT_hbm.at[:, pl.ds(0, 128)],
                    slots_v.at[pl.ds((buf * NSLOT + k) * A, A)],
                    sem,
                ).wait()

        lo = jnp.full((L,), LO, jnp.float32)
        hi = jnp.full((L,), HI, jnp.float32)
        lane = lax.iota(jnp.int32, L)
        zero = jnp.zeros((L,), jnp.int32)

        def compute(g, buf):
            iv = obs_v[g, :]
            incol = lax.rem(iv, 128)
            rowv = (lane + buf * NSLOT) * A
            best = plsc.load_gather(slots_v, [rowv, incol])
            besta = zero
            for a in range(1, A):
                ca = jnp.full((L,), a, jnp.int32)
                va = plsc.load_gather(slots_v, [rowv + ca, incol])
                m = va > best
                best = jnp.where(m, va, best)
                besta = jnp.where(m, ca, besta)
            cols = lane + g * L
            for a in range(A):
                ca = jnp.full((L,), a, jnp.int32)
                vals = jnp.where(besta == ca, hi, lo)
                plsc.store_scatter(out_v, [ca, cols], vals)

        fire(0, 0, sem0)

        def pair_body(p, carry):
            g0 = p * 2
            fire(g0 + 1, 1, sem1)
            drain(0, sem0)
            compute(g0, 0)

            @pl.when(g0 + 2 < n_groups)
            def _():
                fire(g0 + 2, 0, sem0)

            drain(1, sem1)
            compute(g0 + 1, 1)
            return carry

        lax.fori_loop(0, n_pairs, pair_body, 0)
        pltpu.sync_copy(out_v, outT_hbm.at[:, pl.ds(base, b_per_w)])

    return qpolicy(obs, qT).T


# speedup vs baseline: 11.3185x; 1.0114x over previous
import functools

import jax
import jax.numpy as jnp
from jax import lax
from jax.experimental import pallas as pl
from jax.experimental.pallas import tpu as pltpu
from jax.experimental.pallas import tpu_sc as plsc

N_ACTIONS = 18
EPS = 0.99
LO = EPS / N_ACTIONS
HI = 1.0 - EPS + EPS / N_ACTIONS

L = 16
NC, NS = 2, 16
NW = NC * NS
NSLOT = 16


def kernel(obs, q_table):
    B = obs.shape[0]
    A = q_table.shape[1]
    b_per_w = B // NW
    n_groups = b_per_w // L
    n_pairs = n_groups // 2

    qT = q_table.T

    mesh = plsc.VectorSubcoreMesh(core_axis_name="c", subcore_axis_name="s")

    @functools.partial(
        pl.kernel,
        out_type=jax.ShapeDtypeStruct((A, B), jnp.float32),
        mesh=mesh,
        scratch_types=[
            pltpu.VMEM((b_per_w // L, L), jnp.int32),
            pltpu.VMEM((2 * NSLOT * A, 128), jnp.float32),
            pltpu.VMEM((A, b_per_w), jnp.float32),
            pltpu.SemaphoreType.DMA,
            pltpu.SemaphoreType.DMA,
        ],
        compiler_params=pltpu.CompilerParams(
            needs_layout_passes=False, use_tc_tiling_on_sc=True
        ),
    )
    def qpolicy(obs_hbm, qT_hbm, outT_hbm, obs_v, slots_v, out_v,
                sem0, sem1):
        wid = lax.axis_index("s") * NC + lax.axis_index("c")
        base = pl.multiple_of(wid * b_per_w, 128)

        for r in range(b_per_w // L):
            pltpu.sync_copy(
                obs_hbm.at[pl.ds(base + r * L, L)], obs_v.at[r]
            )

        def fire(g, buf, sem):
            iv = obs_v[g, :]
            for k in range(NSLOT):
                i = iv[k]
                tile0 = pl.multiple_of((i // 128) * 128, 128)
                pltpu.async_copy(
                    qT_hbm.at[:, pl.ds(tile0, 128)],
                    slots_v.at[pl.ds((buf * NSLOT + k) * A, A)],
                    sem,
                )

        def drain(buf, sem):
            pltpu.make_async_copy(
                qT_hbm.at[:, pl.ds(0, 128)],
                slots_v.at[pl.ds(buf * NSLOT * A, NSLOT * A)],
                sem,
            ).wait()

        lo = jnp.full((L,), LO, jnp.float32)
        hi = jnp.full((L,), HI, jnp.float32)
        lane = lax.iota(jnp.int32, L)
        zero = jnp.zeros((L,), jnp.int32)

        def compute(g, buf):
            iv = obs_v[g, :]
            incol = lax.rem(iv, 128)
            rowv = (lane + buf * NSLOT) * A
            best = plsc.load_gather(slots_v, [rowv, incol])
            besta = zero
            for a in range(1, A):
                ca = jnp.full((L,), a, jnp.int32)
                va = plsc.load_gather(slots_v, [rowv + ca, incol])
                m = va > best
                best = jnp.where(m, va, best)
                besta = jnp.where(m, ca, besta)
            cols = lane + g * L
            for a in range(A):
                ca = jnp.full((L,), a, jnp.int32)
                vals = jnp.where(besta == ca, hi, lo)
                plsc.store_scatter(out_v, [ca, cols], vals)

        fire(0, 0, sem0)

        def pair_body(p, carry):
            g0 = p * 2
            fire(g0 + 1, 1, sem1)
            drain(0, sem0)
            compute(g0, 0)

            @pl.when(g0 + 2 < n_groups)
            def _():
                fire(g0 + 2, 0, sem0)

            drain(1, sem1)
            compute(g0 + 1, 1)
            return carry

        lax.fori_loop(0, n_pairs, pair_body, 0)
        pltpu.sync_copy(out_v, outT_hbm.at[:, pl.ds(base, b_per_w)])

    return qpolicy(obs, qT).T
